# Initial kernel scaffold; baseline (speedup 1.0000x reference)
import jax
import jax.numpy as jnp
from jax.experimental import pallas as pl


def kernel(x, edge_index, edge_weight, W1, b1, W2, b2):
    raise NotImplementedError("write your pallas kernel here")


# speedup vs baseline: 2.4268x; 2.4268x over previous
import functools

import jax
import jax.numpy as jnp
from jax import lax
from jax.experimental import pallas as pl
from jax.experimental.pallas import tpu as pltpu
from jax.experimental.pallas import tpu_sc as plsc

N_NODES = 10000
N_EDGES = 160000
NFEAT = 256
NHID = 256
NCLASS = 64

NTILES = 16
B = 128
NBLK = 80
EDGES_PER_TILE = NBLK * B
PAD_EDGES = NTILES * EDGES_PER_TILE
CHUNK = 16
ZROWS = 40
COPY_TILES = 10
COPY_ROWS = N_NODES // COPY_TILES


def _make_spmm(D):
    mesh = plsc.VectorSubcoreMesh(core_axis_name="c", subcore_axis_name="s")

    @functools.partial(
        pl.kernel,
        out_type=jax.ShapeDtypeStruct((2 * N_NODES, D), jnp.float32),
        mesh=mesh,
        scratch_types=[
            pltpu.VMEM((CHUNK, 1, B), jnp.int32),
            pltpu.VMEM((CHUNK, 1, B), jnp.int32),
            pltpu.VMEM((CHUNK, 1, B), jnp.float32),
            pltpu.VMEM((B, D), jnp.float32),
            pltpu.VMEM((ZROWS, D), jnp.float32),
            pltpu.VMEM_SHARED((N_NODES, D), jnp.float32),
            pltpu.SemaphoreType.DMA,
        ],
    )
    def spmm(tbl, srcr, dstr, wr, out, srcv, dstv, wv, rows, zbuf, acc, sem):
        c = lax.axis_index("c")
        s = lax.axis_index("s")

        zero16 = jnp.zeros((16,), jnp.float32)

        def zrow(r, carry):
            for f in range(D // 16):
                zbuf[r, pl.ds(f * 16, 16)] = zero16
            return carry

        lax.fori_loop(0, ZROWS, zrow, 0)

        @pl.when(s < COPY_TILES)
        def _zero_acc():
            for k in range(COPY_ROWS // ZROWS):
                pltpu.sync_copy(
                    zbuf, acc.at[pl.ds(s * COPY_ROWS + k * ZROWS, ZROWS)]
                )

        plsc.subcore_barrier()

        def chunk_loop(ch, carry):
            pltpu.sync_copy(srcr.at[c, s, pl.ds(ch * CHUNK, CHUNK)], srcv)
            pltpu.sync_copy(dstr.at[s, pl.ds(ch * CHUNK, CHUNK)], dstv)
            pltpu.sync_copy(wr.at[s, pl.ds(ch * CHUNK, CHUNK)], wv)

            def block(j, inner):
                pltpu.async_copy(tbl.at[srcv.at[j, 0]], rows, sem).wait()

                def group(g, inner2):
                    wvec = wv[j, 0, pl.ds(g * 16, 16)]
                    for l in range(16):
                        w_s = wvec[l]
                        i = g * 16 + l
                        for f in range(D // 16):
                            sl = pl.ds(f * 16, 16)
                            rows[i, sl] = rows[i, sl] * w_s
                    return inner2

                lax.fori_loop(0, B // 16, group, 0)

                pltpu.sync_copy(rows, acc.at[dstv.at[j, 0]], add=True)
                return inner

            lax.fori_loop(0, CHUNK, block, 0)
            return carry

        lax.fori_loop(0, NBLK // CHUNK, chunk_loop, 0)

        plsc.subcore_barrier()

        @pl.when(s < COPY_TILES)
        def _copy_out():
            pltpu.sync_copy(
                acc.at[pl.ds(s * COPY_ROWS, COPY_ROWS)],
                out.at[pl.ds(c * N_NODES + s * COPY_ROWS, COPY_ROWS)],
            )

    return spmm


_spmm128 = _make_spmm(NHID // 2)

_R = 1000
_NRB = N_NODES // _R


def _mm1_body(x_ref, w_ref, o_ref):
    o_ref[...] = jnp.dot(
        x_ref[...], w_ref[...],
        preferred_element_type=jnp.float32,
        precision=lax.Precision.HIGHEST,
    )


def _mm1(x, W1):
    return pl.pallas_call(
        _mm1_body,
        grid=(2, _NRB),
        in_specs=[
            pl.BlockSpec((_R, NFEAT), lambda c, i: (i, 0)),
            pl.BlockSpec((NFEAT, NHID // 2), lambda c, i: (0, c)),
        ],
        out_specs=pl.BlockSpec((_R, NHID // 2), lambda c, i: (c * _NRB + i, 0)),
        out_shape=jax.ShapeDtypeStruct((2 * N_NODES, NHID // 2), jnp.float32),
    )(x, W1)


def _relu_body(h_ref, b_ref, o_ref):
    o_ref[...] = jnp.maximum(h_ref[...] + b_ref[pl.program_id(0)], 0.0)


def _relu_bias(h, b1r):
    return pl.pallas_call(
        _relu_body,
        grid=(2, _NRB),
        in_specs=[
            pl.BlockSpec((_R, NHID // 2), lambda c, i: (c * _NRB + i, 0)),
            pl.BlockSpec((2, NHID // 2), lambda c, i: (0, 0)),
        ],
        out_specs=pl.BlockSpec((_R, NHID // 2), lambda c, i: (c * _NRB + i, 0)),
        out_shape=jax.ShapeDtypeStruct((2 * N_NODES, NHID // 2), jnp.float32),
    )(h, b1r)


def _mmsm_body(a_ref, w_ref, b_ref, o_ref):
    row = (
        jnp.dot(a_ref[0], w_ref[0], preferred_element_type=jnp.float32,
                precision=lax.Precision.HIGHEST)
        + jnp.dot(a_ref[1], w_ref[1], preferred_element_type=jnp.float32,
                  precision=lax.Precision.HIGHEST)
        + b_ref[0]
    )
    m = jnp.max(row, axis=1, keepdims=True)
    e = jnp.exp(row - m)
    o_ref[...] = e / jnp.sum(e, axis=1, keepdims=True)


def _mm2_softmax(agg, W2r, b2r):
    return pl.pallas_call(
        _mmsm_body,
        grid=(_NRB,),
        in_specs=[
            pl.BlockSpec((2, _R, NHID // 2), lambda i: (0, i, 0)),
            pl.BlockSpec((2, NHID // 2, NCLASS), lambda i: (0, 0, 0)),
            pl.BlockSpec((1, NCLASS), lambda i: (0, 0)),
        ],
        out_specs=pl.BlockSpec((_R, NCLASS), lambda i: (i, 0)),
        out_shape=jax.ShapeDtypeStruct((N_NODES, NCLASS), jnp.float32),
    )(agg, W2r, b2r)


def kernel(x, edge_index, edge_weight, W1, b1, W2, b2):
    npad = PAD_EDGES - N_EDGES
    src = jnp.concatenate(
        [edge_index[0].astype(jnp.int32), jnp.zeros((npad,), jnp.int32)]
    )
    dst = jnp.concatenate(
        [edge_index[1].astype(jnp.int32), jnp.zeros((npad,), jnp.int32)]
    )
    w = jnp.concatenate([edge_weight, jnp.zeros((npad,), jnp.float32)])
    src2 = jnp.stack([src, src + N_NODES]).reshape(2, NTILES, NBLK, 1, B)
    dstr = dst.reshape(NTILES, NBLK, 1, B)
    wr = w.reshape(NTILES, NBLK, 1, B)
    W2r = W2.reshape(2, NHID // 2, NCLASS)
    b1r = b1.reshape(2, NHID // 2)
    b2r = b2.reshape(1, NCLASS)

    tbl1 = _mm1(x, W1)
    h = _spmm128(tbl1, src2, dstr, wr)
    hp = _relu_bias(h, b1r)
    agg = _spmm128(hp, src2, dstr, wr)
    return _mm2_softmax(agg.reshape(2, N_NODES, NHID // 2), W2r, b2r)


# speedup vs baseline: 3.1116x; 1.2822x over previous
import functools

import jax
import jax.numpy as jnp
from jax import lax
from jax.experimental import pallas as pl
from jax.experimental.pallas import tpu as pltpu
from jax.experimental.pallas import tpu_sc as plsc

N_NODES = 10000
N_EDGES = 160000
NFEAT = 256
NHID = 256
NCLASS = 64

NTILES = 16
B = 128
NBLK = 80
EDGES_PER_TILE = NBLK * B
PAD_EDGES = NTILES * EDGES_PER_TILE
CHUNK = 16
ZROWS = 40
COPY_TILES = 10
COPY_ROWS = N_NODES // COPY_TILES


def _make_spmm(D):
    mesh = plsc.VectorSubcoreMesh(core_axis_name="c", subcore_axis_name="s")

    @functools.partial(
        pl.kernel,
        out_type=jax.ShapeDtypeStruct((2 * N_NODES, D), jnp.float32),
        mesh=mesh,
        scratch_types=[
            pltpu.VMEM((CHUNK, 1, B), jnp.int32),
            pltpu.VMEM((CHUNK, 1, B), jnp.int32),
            pltpu.VMEM((CHUNK, 1, B), jnp.float32),
            pltpu.VMEM((B, D), jnp.float32),
            pltpu.VMEM((B, D), jnp.float32),
            pltpu.VMEM((ZROWS, D), jnp.float32),
            pltpu.VMEM_SHARED((N_NODES, D), jnp.float32),
            pltpu.SemaphoreType.DMA,
            pltpu.SemaphoreType.DMA,
            pltpu.SemaphoreType.DMA,
            pltpu.SemaphoreType.DMA,
        ],
    )
    def spmm(tbl, srcr, dstr, wr, out,
             srcv, dstv, wv, rows0, rows1, zbuf, acc, g0, g1, s0, s1):
        c = lax.axis_index("c")
        s = lax.axis_index("s")

        zero16 = jnp.zeros((16,), jnp.float32)

        def zrow(r, carry):
            for f in range(D // 16):
                zbuf[r, pl.ds(f * 16, 16)] = zero16
            return carry

        lax.fori_loop(0, ZROWS, zrow, 0)

        @pl.when(s < COPY_TILES)
        def _zero_acc():
            for k in range(COPY_ROWS // ZROWS):
                pltpu.sync_copy(
                    zbuf, acc.at[pl.ds(s * COPY_ROWS + k * ZROWS, ZROWS)]
                )

        plsc.subcore_barrier()

        def scale(rows, j):
            def group(g, inner2):
                wvec = wv[j, 0, pl.ds(g * 16, 16)]
                for l in range(16):
                    w_s = wvec[l]
                    i = g * 16 + l
                    for f in range(D // 16):
                        sl = pl.ds(f * 16, 16)
                        rows[i, sl] = rows[i, sl] * w_s
                return inner2

            lax.fori_loop(0, B // 16, group, 0)

        def chunk_loop(ch, carry):
            pltpu.sync_copy(srcr.at[c, s, pl.ds(ch * CHUNK, CHUNK)], srcv)
            pltpu.sync_copy(dstr.at[s, pl.ds(ch * CHUNK, CHUNK)], dstv)
            pltpu.sync_copy(wr.at[s, pl.ds(ch * CHUNK, CHUNK)], wv)

            pltpu.async_copy(tbl.at[srcv.at[0, 0]], rows0, g0)

            def pair(p, inner):
                j0 = 2 * p
                j1 = 2 * p + 1
                pltpu.async_copy(tbl.at[srcv.at[j1, 0]], rows1, g1)

                pltpu.make_async_copy(tbl.at[srcv.at[j0, 0]], rows0, g0).wait()
                scale(rows0, j0)
                pltpu.async_copy(rows0, acc.at[dstv.at[j0, 0]], s0, add=True)
                pltpu.make_async_copy(rows0, acc.at[dstv.at[j0, 0]], s0).wait()

                @pl.when(p < CHUNK // 2 - 1)
                def _fire_next():
                    pltpu.async_copy(tbl.at[srcv.at[j0 + 2, 0]], rows0, g0)

                pltpu.make_async_copy(tbl.at[srcv.at[j1, 0]], rows1, g1).wait()
                scale(rows1, j1)
                pltpu.async_copy(rows1, acc.at[dstv.at[j1, 0]], s1, add=True)
                pltpu.make_async_copy(rows1, acc.at[dstv.at[j1, 0]], s1).wait()
                return inner

            lax.fori_loop(0, CHUNK // 2, pair, 0)
            return carry

        lax.fori_loop(0, NBLK // CHUNK, chunk_loop, 0)

        plsc.subcore_barrier()

        @pl.when(s < COPY_TILES)
        def _copy_out():
            pltpu.sync_copy(
                acc.at[pl.ds(s * COPY_ROWS, COPY_ROWS)],
                out.at[pl.ds(c * N_NODES + s * COPY_ROWS, COPY_ROWS)],
            )

    return spmm


_spmm128 = _make_spmm(NHID // 2)

_R = 1000
_NRB = N_NODES // _R


def _mm1_body(x_ref, w_ref, o_ref):
    o_ref[...] = jnp.dot(
        x_ref[...], w_ref[...],
        preferred_element_type=jnp.float32,
        precision=lax.Precision.HIGHEST,
    )


def _mm1(x, W1):
    return pl.pallas_call(
        _mm1_body,
        grid=(2, _NRB),
        in_specs=[
            pl.BlockSpec((_R, NFEAT), lambda c, i: (i, 0)),
            pl.BlockSpec((NFEAT, NHID // 2), lambda c, i: (0, c)),
        ],
        out_specs=pl.BlockSpec((_R, NHID // 2), lambda c, i: (c * _NRB + i, 0)),
        out_shape=jax.ShapeDtypeStruct((2 * N_NODES, NHID // 2), jnp.float32),
    )(x, W1)


def _relu_body(h_ref, b_ref, o_ref):
    o_ref[...] = jnp.maximum(h_ref[...] + b_ref[pl.program_id(0)], 0.0)


def _relu_bias(h, b1r):
    return pl.pallas_call(
        _relu_body,
        grid=(2, _NRB),
        in_specs=[
            pl.BlockSpec((_R, NHID // 2), lambda c, i: (c * _NRB + i, 0)),
            pl.BlockSpec((2, NHID // 2), lambda c, i: (0, 0)),
        ],
        out_specs=pl.BlockSpec((_R, NHID // 2), lambda c, i: (c * _NRB + i, 0)),
        out_shape=jax.ShapeDtypeStruct((2 * N_NODES, NHID // 2), jnp.float32),
    )(h, b1r)


def _mmsm_body(a_ref, w_ref, b_ref, o_ref):
    row = (
        jnp.dot(a_ref[0], w_ref[0], preferred_element_type=jnp.float32,
                precision=lax.Precision.HIGHEST)
        + jnp.dot(a_ref[1], w_ref[1], preferred_element_type=jnp.float32,
                  precision=lax.Precision.HIGHEST)
        + b_ref[0]
    )
    m = jnp.max(row, axis=1, keepdims=True)
    e = jnp.exp(row - m)
    o_ref[...] = e / jnp.sum(e, axis=1, keepdims=True)


def _mm2_softmax(agg, W2r, b2r):
    return pl.pallas_call(
        _mmsm_body,
        grid=(_NRB,),
        in_specs=[
            pl.BlockSpec((2, _R, NHID // 2), lambda i: (0, i, 0)),
            pl.BlockSpec((2, NHID // 2, NCLASS), lambda i: (0, 0, 0)),
            pl.BlockSpec((1, NCLASS), lambda i: (0, 0)),
        ],
        out_specs=pl.BlockSpec((_R, NCLASS), lambda i: (i, 0)),
        out_shape=jax.ShapeDtypeStruct((N_NODES, NCLASS), jnp.float32),
    )(agg, W2r, b2r)


def kernel(x, edge_index, edge_weight, W1, b1, W2, b2):
    npad = PAD_EDGES - N_EDGES
    src = jnp.concatenate(
        [edge_index[0].astype(jnp.int32), jnp.zeros((npad,), jnp.int32)]
    )
    dst = jnp.concatenate(
        [edge_index[1].astype(jnp.int32), jnp.zeros((npad,), jnp.int32)]
    )
    w = jnp.concatenate([edge_weight, jnp.zeros((npad,), jnp.float32)])
    src2 = jnp.stack([src, src + N_NODES]).reshape(2, NTILES, NBLK, 1, B)
    dstr = dst.reshape(NTILES, NBLK, 1, B)
    wr = w.reshape(NTILES, NBLK, 1, B)
    W2r = W2.reshape(2, NHID // 2, NCLASS)
    b1r = b1.reshape(2, NHID // 2)
    b2r = b2.reshape(1, NCLASS)

    tbl1 = _mm1(x, W1)
    h = _spmm128(tbl1, src2, dstr, wr)
    hp = _relu_bias(h, b1r)
    agg = _spmm128(hp, src2, dstr, wr)
    return _mm2_softmax(agg.reshape(2, N_NODES, NHID // 2), W2r, b2r)
